# Initial kernel scaffold; baseline (speedup 1.0000x reference)
import jax
import jax.numpy as jnp
from jax.experimental import pallas as pl


def kernel(tokens, table):
    raise NotImplementedError("write your pallas kernel here")


# speedup vs baseline: 16.6110x; 16.6110x over previous
import functools

import jax
import jax.numpy as jnp
from jax import lax
from jax.experimental import pallas as pl
from jax.experimental.pallas import tpu as pltpu
from jax.experimental.pallas import tpu_sc as plsc

B = 16384
L = 48
V = 128
D = 64
NC = 2
NS = 16
NW = NC * NS
RPW = B // NW

_MESH = plsc.VectorSubcoreMesh(
    core_axis_name="c", subcore_axis_name="s", num_cores=NC, num_subcores=NS
)


@functools.partial(
    pl.kernel,
    out_type=jax.ShapeDtypeStruct((B * D,), jnp.float32),
    mesh=_MESH,
    compiler_params=pltpu.CompilerParams(needs_layout_passes=False),
    scratch_types=[
        pltpu.VMEM((V * D,), jnp.float32),
        pltpu.VMEM((RPW * L,), jnp.int32),
        pltpu.VMEM((RPW * D,), jnp.float32),
    ],
)
def _encode(tokens_hbm, table_hbm, out_hbm, table_v, tok_v, out_v):
    wid = lax.axis_index("s") * NC + lax.axis_index("c")
    base = wid * RPW

    pltpu.sync_copy(table_hbm, table_v)
    pltpu.sync_copy(tokens_hbm.at[pl.ds(base * L, RPW * L)], tok_v)

    zeros = jnp.zeros((16,), jnp.float32)
    for j in range(D // 16):
        table_v[pl.ds(16 * j, 16)] = zeros

    cols = [lax.iota(jnp.int32, 16) + 16 * j for j in range(D // 16)]
    splat_idx = [jnp.full((16, 1), i, jnp.int32) for i in range(16)]
    dnums = lax.GatherDimensionNumbers(
        offset_dims=(), collapsed_slice_dims=(0,), start_index_map=(0,)
    )

    def lane_perm(v, idx):
        return lax.gather(
            v,
            idx,
            dimension_numbers=dnums,
            slice_sizes=(1,),
            mode=lax.GatherScatterMode.PROMISE_IN_BOUNDS,
        )

    def lane_splat(tv, i):
        return lane_perm(tv, splat_idx[i])

    lane_iota = lax.iota(jnp.int32, 16)
    xor_idx = [(lane_iota ^ s).reshape(16, 1) for s in (8, 4, 2, 1)]

    def lane_reduce_sum(v):
        for idx in xor_idx:
            v = v + lane_perm(v, idx)
        return v
    ione = jnp.full((16,), 1, jnp.int32)
    izero = jnp.full((16,), 0, jnp.int32)

    def row_body(r, carry):
        tvs = [tok_v[pl.ds(r * L + 16 * k, 16)] for k in range(L // 16)]
        nz_vec = izero
        for tv in tvs:
            nz_vec = nz_vec + jnp.where(tv != 0, ione, izero)
        cnt = jnp.maximum(lane_reduce_sum(nz_vec), ione)
        rcp = 1.0 / cnt.astype(jnp.float32)

        accs = [zeros, zeros, zeros, zeros]
        for tv in tvs:
            tv64 = tv * D
            for i in range(16):
                sp = lane_splat(tv64, i)
                for j in range(D // 16):
                    accs[j] = accs[j] + plsc.load_gather(table_v, [sp + cols[j]])
        for j in range(D // 16):
            out_v[pl.ds(r * D + 16 * j, 16)] = accs[j] * rcp
        return carry

    lax.fori_loop(0, RPW, row_body, 0)
    pltpu.sync_copy(out_v, out_hbm.at[pl.ds(base * D, RPW * D)])


def kernel(tokens, table):
    out = _encode(tokens.reshape(B * L), table.reshape(V * D))
    return out.reshape(B, D)


# speedup vs baseline: 31.8849x; 1.9195x over previous
import functools

import jax
import jax.numpy as jnp
from jax import lax
from jax.experimental import pallas as pl
from jax.experimental.pallas import tpu as pltpu
from jax.experimental.pallas import tpu_sc as plsc

B = 16384
L = 48
V = 128
D = 64
PW = D // 2
NC = 2
NS = 16
NW = NC * NS
RPW = B // NW

_MESH = plsc.VectorSubcoreMesh(
    core_axis_name="c", subcore_axis_name="s", num_cores=NC, num_subcores=NS
)


@functools.partial(
    pl.kernel,
    out_type=jax.ShapeDtypeStruct((B * D,), jnp.float32),
    mesh=_MESH,
    compiler_params=pltpu.CompilerParams(needs_layout_passes=False),
    scratch_types=[
        pltpu.VMEM((V * PW,), jnp.int32),
        pltpu.VMEM((RPW * L,), jnp.int32),
        pltpu.VMEM((RPW * D,), jnp.float32),
    ],
)
def _encode(tokens_hbm, table_hbm, out_hbm, table_v, tok_v, out_v):
    wid = lax.axis_index("s") * NC + lax.axis_index("c")
    base = wid * RPW

    pltpu.sync_copy(table_hbm, table_v)
    pltpu.sync_copy(tokens_hbm.at[pl.ds(base * L, RPW * L)], tok_v)

    izeros = jnp.zeros((16,), jnp.int32)
    for j in range(PW // 16):
        table_v[pl.ds(16 * j, 16)] = izeros

    lane_iota = lax.iota(jnp.int32, 16)
    cols = [lane_iota + 16 * j for j in range(PW // 16)]
    splat_idx = [jnp.full((16, 1), i, jnp.int32) for i in range(16)]
    dnums = lax.GatherDimensionNumbers(
        offset_dims=(), collapsed_slice_dims=(0,), start_index_map=(0,)
    )

    def lane_perm(v, idx):
        return lax.gather(
            v,
            idx,
            dimension_numbers=dnums,
            slice_sizes=(1,),
            mode=lax.GatherScatterMode.PROMISE_IN_BOUNDS,
        )

    xor_idx = [(lane_iota ^ s).reshape(16, 1) for s in (8, 4, 2, 1)]

    def lane_reduce_sum(v):
        for idx in xor_idx:
            v = v + lane_perm(v, idx)
        return v

    ione = jnp.full((16,), 1, jnp.int32)
    himask = jnp.full((16,), -65536, jnp.int32)
    bzeros = plsc.bitcast(izeros, jnp.bfloat16)

    def row_body(r, carry):
        tvs = [tok_v[pl.ds(r * L + 16 * k, 16)] for k in range(L // 16)]
        nz_vec = izeros
        for tv in tvs:
            nz_vec = nz_vec + jnp.where(tv != 0, ione, izeros)
        cnt = jnp.maximum(lane_reduce_sum(nz_vec), ione)
        rcp = 1.0 / cnt.astype(jnp.float32)

        group_accs = []
        for tv in tvs:
            tvp = tv * PW
            accs = [bzeros, bzeros]
            for i in range(16):
                sp = lane_perm(tvp, splat_idx[i])
                for j in range(PW // 16):
                    g = plsc.load_gather(table_v, [sp + cols[j]])
                    accs[j] = accs[j] + plsc.bitcast(g, jnp.bfloat16)
            group_accs.append(accs)
        for j in range(PW // 16):
            tot = (group_accs[0][j] + group_accs[1][j]) + group_accs[2][j]
            ti = plsc.bitcast(tot, jnp.int32)
            lo = plsc.bitcast(lax.shift_left(ti, 16), jnp.float32)
            hi = plsc.bitcast(ti & himask, jnp.float32)
            out_v[pl.ds(r * D + 32 * j, 16)] = lo * rcp
            out_v[pl.ds(r * D + 32 * j + 16, 16)] = hi * rcp
        return carry

    lax.fori_loop(0, RPW, row_body, 0)
    pltpu.sync_copy(out_v, out_hbm.at[pl.ds(base * D, RPW * D)])


def _pack_table(table):
    tb = table.astype(jnp.bfloat16)
    u = lax.bitcast_convert_type(tb, jnp.uint16).reshape(V, 2, 2, 16)
    lo = u[:, :, 0, :].astype(jnp.uint32)
    hi = u[:, :, 1, :].astype(jnp.uint32)
    packed = (hi << 16) | lo
    return lax.bitcast_convert_type(packed, jnp.int32).reshape(V * PW)


def kernel(tokens, table):
    out = _encode(tokens.reshape(B * L), _pack_table(table))
    return out.reshape(B, D)
